# baseline (speedup 1.0000x reference)
import functools

import jax
import jax.numpy as jnp
from jax import lax
from jax.experimental import pallas as pl
from jax.experimental.pallas import tpu as pltpu
from jax.experimental.pallas import tpu_sc as plsc

VOCAB = 1000000
EMBED = 64
BATCH = 16384
CTX = 20

NC, NS = 2, 16
NW = NC * NS
BPW = BATCH // NW
CHUNK = 16
NCHUNK = BPW // CHUNK
IDX_PER_GATHER = 80
NGATHER = (CHUNK * CTX) // IDX_PER_GATHER
NREG = EMBED // 16


def _cbow_body(ctx_hbm, cen_hbm, u_hbm, v_hbm, out_hbm,
               idx_v, cidx_v, u_buf, c_buf, res_buf, sem):
    wid = lax.axis_index("s") * NC + lax.axis_index("c")
    pltpu.sync_copy(ctx_hbm.at[wid], idx_v)
    pltpu.sync_copy(cen_hbm.at[wid], cidx_v)

    lane = lax.iota(jnp.int32, 16)
    inv_ctx = jnp.float32(1.0 / CTX)

    def chunk_body(g, carry):
        cps = [
            pltpu.async_copy(u_hbm.at[idx_v.at[g, k]], u_buf.at[k], sem)
            for k in range(NGATHER)
        ]
        cps.append(pltpu.async_copy(v_hbm.at[cidx_v.at[g]], c_buf, sem))
        for cp in cps:
            cp.wait()

        resv = jnp.zeros((16,), jnp.float32)
        for r in range(CHUNK):
            c_regs = [c_buf[r, pl.ds(t * 16, 16)] for t in range(NREG)]
            accs = [None] * NREG
            for j in range(CTX):
                flat = r * CTX + j
                k, row = flat // IDX_PER_GATHER, flat % IDX_PER_GATHER
                for t in range(NREG):
                    prod = u_buf[k, row, pl.ds(t * 16, 16)] * c_regs[t]
                    accs[t] = prod if accs[t] is None else accs[t] + prod
            p = (accs[0] + accs[1]) + (accs[2] + accs[3])
            s = jnp.sum(p) * inv_ctx
            resv = jnp.where(lane == r, s, resv)
        res_buf[g, :] = resv
        return carry

    lax.fori_loop(0, NCHUNK, chunk_body, 0)
    pltpu.sync_copy(res_buf, out_hbm.at[wid])


@jax.jit
def _cbow(ctx_r, cen_r, embed_u, embed_v):
    mesh = plsc.VectorSubcoreMesh(core_axis_name="c", subcore_axis_name="s",
                                  num_cores=NC, num_subcores=NS)
    f = pl.kernel(
        _cbow_body,
        out_type=jax.ShapeDtypeStruct((NW, NCHUNK, CHUNK), jnp.float32),
        mesh=mesh,
        scratch_types=[
            pltpu.VMEM((NCHUNK, NGATHER, IDX_PER_GATHER), jnp.int32),
            pltpu.VMEM((NCHUNK, CHUNK), jnp.int32),
            pltpu.VMEM((NGATHER, IDX_PER_GATHER, EMBED), jnp.float32),
            pltpu.VMEM((CHUNK, EMBED), jnp.float32),
            pltpu.VMEM((NCHUNK, CHUNK), jnp.float32),
            pltpu.SemaphoreType.DMA,
        ],
        compiler_params=pltpu.CompilerParams(needs_layout_passes=False,
                                             use_tc_tiling_on_sc=False),
    )
    return f(ctx_r, cen_r, embed_u, embed_v)


def kernel(contexts, center, embed_u, embed_v):
    ctx_r = jnp.asarray(contexts, jnp.int32).reshape(
        NW, NCHUNK, NGATHER, IDX_PER_GATHER)
    cen_r = jnp.asarray(center, jnp.int32).reshape(NW, NCHUNK, CHUNK)
    out = _cbow(ctx_r, cen_r, embed_u, embed_v)
    return out.reshape(BATCH, 1, 1)


# speedup vs baseline: 1.2504x; 1.2504x over previous
import functools

import jax
import jax.numpy as jnp
from jax import lax
from jax.experimental import pallas as pl
from jax.experimental.pallas import tpu as pltpu
from jax.experimental.pallas import tpu_sc as plsc

VOCAB = 1000000
EMBED = 64
BATCH = 16384
CTX = 20

NC, NS = 2, 16
NW = NC * NS
BPW = BATCH // NW
CHUNK = 16
NCHUNK = BPW // CHUNK
NREG = EMBED // 16


def _cbow_body(ctx_hbm, cen_hbm, u_hbm, v_hbm, out_hbm,
               vidx, vcen, u_buf, c_buf, res_buf, sem):
    wid = lax.axis_index("s") * NC + lax.axis_index("c")

    lane = lax.iota(jnp.int32, 16)
    inv_ctx = jnp.float32(1.0 / CTX)

    def chunk_body(g, carry):
        cbase = wid * NCHUNK + g
        pltpu.sync_copy(ctx_hbm.at[pl.ds(cbase * (CHUNK * CTX), CHUNK * CTX)],
                        vidx)
        pltpu.sync_copy(cen_hbm.at[pl.ds(cbase * CHUNK, CHUNK)], vcen)
        cps = []
        cvec = vcen[...]
        jvecs = [vidx[pl.ds(j * CHUNK, CHUNK)] for j in range(CTX)]
        for r in range(CHUNK):
            for j in range(CTX):
                cps.append(pltpu.async_copy(
                    u_hbm.at[jvecs[j][r]], u_buf.at[r * CTX + j], sem))
            cps.append(pltpu.async_copy(
                v_hbm.at[cvec[r]], c_buf.at[r], sem))
        for cp in cps:
            cp.wait()

        resv = jnp.zeros((16,), jnp.float32)
        for r in range(CHUNK):
            c_regs = [c_buf[r, pl.ds(t * 16, 16)] for t in range(NREG)]
            accs = [None] * NREG
            for j in range(CTX):
                f = r * CTX + j
                for t in range(NREG):
                    prod = u_buf[f, pl.ds(t * 16, 16)] * c_regs[t]
                    accs[t] = prod if accs[t] is None else accs[t] + prod
            p = (accs[0] + accs[1]) + (accs[2] + accs[3])
            s = jnp.sum(p) * inv_ctx
            resv = jnp.where(lane == r, s, resv)
        res_buf[pl.ds(g * CHUNK, CHUNK)] = resv
        return carry

    lax.fori_loop(0, NCHUNK, chunk_body, 0)
    pltpu.sync_copy(res_buf, out_hbm.at[pl.ds(wid * BPW, BPW)])


@jax.jit
def _cbow(ctx_r, cen_r, embed_u, embed_v):
    mesh = plsc.VectorSubcoreMesh(core_axis_name="c", subcore_axis_name="s",
                                  num_cores=NC, num_subcores=NS)
    f = pl.kernel(
        _cbow_body,
        out_type=jax.ShapeDtypeStruct((BATCH,), jnp.float32),
        mesh=mesh,
        scratch_types=[
            pltpu.VMEM((CHUNK * CTX,), jnp.int32),
            pltpu.VMEM((CHUNK,), jnp.int32),
            pltpu.VMEM((CHUNK * CTX, EMBED), jnp.float32),
            pltpu.VMEM((CHUNK, EMBED), jnp.float32),
            pltpu.VMEM((BPW,), jnp.float32),
            pltpu.SemaphoreType.DMA,
        ],
        compiler_params=pltpu.CompilerParams(needs_layout_passes=False,
                                             use_tc_tiling_on_sc=True),
    )
    return f(ctx_r, cen_r, embed_u, embed_v)


def kernel(contexts, center, embed_u, embed_v):
    ctx_r = jnp.asarray(contexts, jnp.int32).reshape(
        NW, NCHUNK, CHUNK, CTX).transpose(0, 1, 3, 2).reshape(BATCH * CTX)
    cen_r = jnp.asarray(center, jnp.int32).reshape(BATCH)
    out = _cbow(ctx_r, cen_r, embed_u, embed_v)
    return out.reshape(BATCH, 1, 1)


# speedup vs baseline: 1.3294x; 1.0632x over previous
import functools

import jax
import jax.numpy as jnp
from jax import lax
from jax.experimental import pallas as pl
from jax.experimental.pallas import tpu as pltpu
from jax.experimental.pallas import tpu_sc as plsc

VOCAB = 1000000
EMBED = 64
BATCH = 16384
CTX = 20

NC, NS = 2, 16
NW = NC * NS
BPW = BATCH // NW
CHUNK = 16
NCHUNK = BPW // CHUNK
NREG = EMBED // 16
IPW = BPW * CTX


def _cbow_body(ctx_hbm, cen_hbm, u_hbm, v_hbm, out_hbm,
               vidx, vcen, u_buf, c_buf, res_buf, sem):
    wid = lax.axis_index("s") * NC + lax.axis_index("c")

    pltpu.sync_copy(ctx_hbm.at[pl.ds(wid * IPW, IPW)], vidx)
    pltpu.sync_copy(cen_hbm.at[pl.ds(wid * BPW, BPW)], vcen)

    lane = lax.iota(jnp.int32, 16)
    inv_ctx = jnp.float32(1.0 / CTX)

    def chunk_body(g, carry):
        cvec = vcen[pl.ds(g * CHUNK, CHUNK)]
        jvecs = [vidx[pl.ds(g * (CHUNK * CTX) + j * CHUNK, CHUNK)]
                 for j in range(CTX)]
        for r in range(CHUNK):
            for j in range(CTX):
                pltpu.async_copy(
                    u_hbm.at[jvecs[j][r]], u_buf.at[r * CTX + j], sem)
            pltpu.async_copy(v_hbm.at[cvec[r]], c_buf.at[r], sem)
        pltpu.make_async_copy(
            u_hbm.at[pl.ds(0, CHUNK * CTX)], u_buf, sem).wait()
        pltpu.make_async_copy(
            v_hbm.at[pl.ds(0, CHUNK)], c_buf, sem).wait()

        resv = jnp.zeros((16,), jnp.float32)
        for r in range(CHUNK):
            c_regs = [c_buf[r, pl.ds(t * 16, 16)] for t in range(NREG)]
            accs = [None] * NREG
            for j in range(CTX):
                f = r * CTX + j
                for t in range(NREG):
                    prod = u_buf[f, pl.ds(t * 16, 16)] * c_regs[t]
                    accs[t] = prod if accs[t] is None else accs[t] + prod
            p = (accs[0] + accs[1]) + (accs[2] + accs[3])
            s = jnp.sum(p) * inv_ctx
            resv = jnp.where(lane == r, s, resv)
        res_buf[pl.ds(g * CHUNK, CHUNK)] = resv
        return carry

    lax.fori_loop(0, NCHUNK, chunk_body, 0)
    pltpu.sync_copy(res_buf, out_hbm.at[pl.ds(wid * BPW, BPW)])


@jax.jit
def _cbow(ctx_r, cen_r, embed_u, embed_v):
    mesh = plsc.VectorSubcoreMesh(core_axis_name="c", subcore_axis_name="s",
                                  num_cores=NC, num_subcores=NS)
    f = pl.kernel(
        _cbow_body,
        out_type=jax.ShapeDtypeStruct((BATCH,), jnp.float32),
        mesh=mesh,
        scratch_types=[
            pltpu.VMEM((IPW,), jnp.int32),
            pltpu.VMEM((BPW,), jnp.int32),
            pltpu.VMEM((CHUNK * CTX, EMBED), jnp.float32),
            pltpu.VMEM((CHUNK, EMBED), jnp.float32),
            pltpu.VMEM((BPW,), jnp.float32),
            pltpu.SemaphoreType.DMA,
        ],
        compiler_params=pltpu.CompilerParams(needs_layout_passes=False,
                                             use_tc_tiling_on_sc=True),
    )
    return f(ctx_r, cen_r, embed_u, embed_v)


def kernel(contexts, center, embed_u, embed_v):
    ctx_r = jnp.asarray(contexts, jnp.int32).reshape(
        NW, NCHUNK, CHUNK, CTX).transpose(0, 1, 3, 2).reshape(BATCH * CTX)
    cen_r = jnp.asarray(center, jnp.int32).reshape(BATCH)
    out = _cbow(ctx_r, cen_r, embed_u, embed_v)
    return out.reshape(BATCH, 1, 1)
